# baseline (speedup 1.0000x reference)
import functools

import jax
import jax.numpy as jnp
from jax import lax
from jax.experimental import pallas as pl
from jax.experimental.pallas import tpu as pltpu
from jax.experimental.pallas import tpu_sc as plsc

_C = 19
_SMOOTHING = 0.1
_CONFIDENCE = 1.0 - _SMOOTHING
_SV = _SMOOTHING / (_C - 1)
_IGNORE = 255

_B = 8
_P = 512 * 512
_NW = 32
_CH = 2048
_CHUNKS_PER_BATCH = _P // _CH
_TOTAL_CHUNKS = _B * _CHUNKS_PER_BATCH
_CHUNKS_PER_W = _TOTAL_CHUNKS // _NW
_GROUPS = _CH // 16

_LN2 = 0.6931471805599453
_SQRT2 = 1.4142135623730951


def _log_f32(s):
    bits = lax.bitcast_convert_type(s, jnp.int32)
    e = lax.shift_right_logical(bits, 23) - 127
    mant = lax.bitwise_or(lax.bitwise_and(bits, 0x007FFFFF), 0x3F800000)
    m = lax.bitcast_convert_type(mant, jnp.float32)
    big = m > _SQRT2
    m = jnp.where(big, m * 0.5, m)
    ef = e.astype(jnp.float32) + jnp.where(big, 1.0, 0.0)
    z = (m - 1.0) / (m + 1.0)
    z2 = z * z
    logm = z * (2.0 + z2 * (2.0 / 3.0 + z2 * (2.0 / 5.0 + z2 * (2.0 / 7.0))))
    return ef * _LN2 + logm


def _body(pred_hbm, tgt_hbm, out_hbm, x_vmem, t_vmem, acc_vmem):
    wid = lax.axis_index("s") * 2 + lax.axis_index("c")
    lane = lax.iota(jnp.int32, 16)

    def chunk_body(j, acc):
        g = wid * _CHUNKS_PER_W + j
        b = g // _CHUNKS_PER_BATCH
        p0 = (g % _CHUNKS_PER_BATCH) * _CH
        pltpu.sync_copy(pred_hbm.at[b, :, pl.ds(p0, _CH)], x_vmem)
        pltpu.sync_copy(tgt_hbm.at[b, pl.ds(p0, _CH)], t_vmem)

        def grp_body(i, acc):
            base = i * 16
            xs = [x_vmem[c, pl.ds(base, 16)] for c in range(_C)]
            m = xs[0]
            sp = xs[0]
            for c in range(1, _C):
                m = jnp.maximum(m, xs[c])
                sp = sp + xs[c]
            s = jnp.exp(xs[0] - m)
            for c in range(1, _C):
                s = s + jnp.exp(xs[c] - m)
            lse = m + _log_f32(s)
            t = t_vmem[pl.ds(base, 16)]
            mask = t != _IGNORE
            tc = jnp.where(mask, t, 0)
            pt = plsc.load_gather(x_vmem, [tc, base + lane])
            val = lse - _SV * sp - (_CONFIDENCE - _SV) * pt
            return acc + jnp.where(mask, val, 0.0)

        return lax.fori_loop(0, _GROUPS, grp_body, acc)

    acc = lax.fori_loop(0, _CHUNKS_PER_W, chunk_body,
                        jnp.zeros((16,), jnp.float32))
    acc_vmem[...] = acc
    pltpu.sync_copy(acc_vmem, out_hbm.at[wid])


@jax.jit
def kernel(pred, target):
    pred3 = pred.reshape(_B, _C, _P)
    tgt2 = target.reshape(_B, _P)
    mesh = plsc.VectorSubcoreMesh(core_axis_name="c", subcore_axis_name="s")
    partials = pl.kernel(
        _body,
        out_type=jax.ShapeDtypeStruct((_NW, 16), jnp.float32),
        mesh=mesh,
        scratch_types=[
            pltpu.VMEM((_C, _CH), jnp.float32),
            pltpu.VMEM((_CH,), jnp.int32),
            pltpu.VMEM((16,), jnp.float32),
        ],
        compiler_params=pltpu.CompilerParams(needs_layout_passes=False),
    )(pred3, tgt2)
    return jnp.sum(partials) * (1.0 / (_B * _P))


# speedup vs baseline: 1.2721x; 1.2721x over previous
import functools

import jax
import jax.numpy as jnp
from jax import lax
from jax.experimental import pallas as pl
from jax.experimental.pallas import tpu as pltpu
from jax.experimental.pallas import tpu_sc as plsc

_C = 19
_SMOOTHING = 0.1
_CONFIDENCE = 1.0 - _SMOOTHING
_SV = _SMOOTHING / (_C - 1)
_IGNORE = 255

_B = 8
_P = 512 * 512
_NW = 32
_CH = 2048
_CHUNKS_PER_BATCH = _P // _CH
_TOTAL_CHUNKS = _B * _CHUNKS_PER_BATCH
_CHUNKS_PER_W = _TOTAL_CHUNKS // _NW
_GROUPS = _CH // 16

_LN2 = 0.6931471805599453
_SQRT2 = 1.4142135623730951


def _log_f32(s):
    bits = lax.bitcast_convert_type(s, jnp.int32)
    e = lax.shift_right_logical(bits, 23) - 127
    mant = lax.bitwise_or(lax.bitwise_and(bits, 0x007FFFFF), 0x3F800000)
    m = lax.bitcast_convert_type(mant, jnp.float32)
    big = m > _SQRT2
    m = jnp.where(big, m * 0.5, m)
    ef = e.astype(jnp.float32) + jnp.where(big, 1.0, 0.0)
    z = (m - 1.0) / (m + 1.0)
    z2 = z * z
    logm = z * (2.0 + z2 * (2.0 / 3.0 + z2 * (2.0 / 5.0 + z2 * (2.0 / 7.0))))
    return ef * _LN2 + logm


def _body(pred_hbm, tgt_hbm, out_hbm, x_vmem, t_vmem, acc_vmem, psem, tsem):
    wid = lax.axis_index("s") * 2 + lax.axis_index("c")
    lane = lax.iota(jnp.int32, 16)

    def start_fetch(g, slot):
        gg = jnp.minimum(g, _CHUNKS_PER_W - 1)
        gc = wid * _CHUNKS_PER_W + gg
        b = gc // _CHUNKS_PER_BATCH
        p0 = (gc % _CHUNKS_PER_BATCH) * _CH
        pltpu.make_async_copy(
            pred_hbm.at[b, :, pl.ds(p0, _CH)], x_vmem.at[slot],
            psem.at[slot]).start()
        pltpu.make_async_copy(
            tgt_hbm.at[b, pl.ds(p0, _CH)], t_vmem.at[slot],
            tsem.at[slot]).start()

    def wait_fetch(slot):
        pltpu.make_async_copy(
            pred_hbm.at[0, :, pl.ds(0, _CH)], x_vmem.at[slot],
            psem.at[slot]).wait()
        pltpu.make_async_copy(
            tgt_hbm.at[0, pl.ds(0, _CH)], t_vmem.at[slot],
            tsem.at[slot]).wait()

    def compute(slot, acc):
        def grp_body(i, acc):
            base = i * 16
            xs = [x_vmem[slot, c, pl.ds(base, 16)] for c in range(_C)]
            m = xs[0]
            sp = xs[0]
            for c in range(1, _C):
                m = jnp.maximum(m, xs[c])
                sp = sp + xs[c]
            s = jnp.exp(xs[0] - m)
            for c in range(1, _C):
                s = s + jnp.exp(xs[c] - m)
            lse = m + _log_f32(s)
            t = t_vmem[slot, pl.ds(base, 16)]
            mask = t != _IGNORE
            tc = jnp.where(mask, t, 0)
            pt = plsc.load_gather(x_vmem.at[slot], [tc, base + lane])
            val = lse - _SV * sp - (_CONFIDENCE - _SV) * pt
            return acc + jnp.where(mask, val, 0.0)

        return lax.fori_loop(0, _GROUPS, grp_body, acc)

    start_fetch(0, 0)
    start_fetch(1, 1)

    def chunk_body(j, acc):
        for slot in range(2):
            g = j * 2 + slot
            wait_fetch(slot)
            acc = compute(slot, acc)
            start_fetch(g + 2, slot)
        return acc

    acc = lax.fori_loop(0, _CHUNKS_PER_W // 2, chunk_body,
                        jnp.zeros((16,), jnp.float32))
    wait_fetch(0)
    wait_fetch(1)
    acc_vmem[...] = acc
    pltpu.sync_copy(acc_vmem, out_hbm.at[wid])


@jax.jit
def kernel(pred, target):
    pred3 = pred.reshape(_B, _C, _P)
    tgt2 = target.reshape(_B, _P)
    mesh = plsc.VectorSubcoreMesh(core_axis_name="c", subcore_axis_name="s")
    partials = pl.kernel(
        _body,
        out_type=jax.ShapeDtypeStruct((_NW, 16), jnp.float32),
        mesh=mesh,
        scratch_types=[
            pltpu.VMEM((2, _C, _CH), jnp.float32),
            pltpu.VMEM((2, _CH), jnp.int32),
            pltpu.VMEM((16,), jnp.float32),
            pltpu.SemaphoreType.DMA((2,)),
            pltpu.SemaphoreType.DMA((2,)),
        ],
        compiler_params=pltpu.CompilerParams(needs_layout_passes=False),
    )(pred3, tgt2)
    return jnp.sum(partials) * (1.0 / (_B * _P))
